# baseline (speedup 1.0000x reference)
import jax
import jax.numpy as jnp
from jax.experimental import pallas as pl
from jax.experimental.pallas import tpu as pltpu

N = 4096
L = 16384
R = 512
C = 2048
NBLK = N // R
CBLK = L // C


def _softplus(x):
    return jnp.maximum(x, 0.0) + jnp.log1p(jnp.exp(-jnp.abs(x)))


def _loss_kernel(p3_ref, pf_ref, gt_ref, gtT_ref, predT_ref, out_ref):
    i = pl.program_id(0)

    p3 = p3_ref[...]
    a2 = jnp.sum(p3 * p3, axis=1, keepdims=True)

    def nn_body(c, carry):
        run_min, run_vals = carry
        g3 = gtT_ref[:, pl.ds(c * C, C)]
        ab = jax.lax.dot_general(p3, g3, (((1,), (0,)), ((), ())),
                                 preferred_element_type=jnp.float32)
        b2 = jnp.sum(g3 * g3, axis=0, keepdims=True)
        sq = a2 + b2 - 2.0 * ab
        sqc = jnp.maximum(sq, 1e-12)
        bmin = jnp.min(sqc, axis=1, keepdims=True)
        col = jax.lax.broadcasted_iota(jnp.int32, (R, C), 1)
        bidx = jnp.min(jnp.where(sqc <= bmin, col, C), axis=1,
                       keepdims=True)
        onehot = (col == bidx).astype(jnp.float32)
        g6 = gt_ref[pl.ds(c * C, C), :]
        wvals = jax.lax.dot_general(onehot, g6, (((1,), (0,)), ((), ())),
                                    preferred_element_type=jnp.float32,
                                    precision=jax.lax.Precision.HIGHEST)
        better = bmin < run_min
        return (jnp.where(better, bmin, run_min),
                jnp.where(better, wvals, run_vals))

    init = (jnp.full((R, 1), jnp.inf, jnp.float32),
            jnp.zeros((R, 6), jnp.float32))
    _, closest = jax.lax.fori_loop(0, CBLK, nn_body, init)

    diff = p3 - closest[:, 0:3]
    attr = jnp.sum(diff * diff)

    pn = pf_ref[:, 3:6]
    pnu = pn / jnp.maximum(jnp.sqrt(jnp.sum(pn * pn, axis=1, keepdims=True)),
                           1e-5)
    gn = closest[:, 3:6]
    gnu = gn / jnp.maximum(jnp.sqrt(jnp.sum(gn * gn, axis=1, keepdims=True)),
                           1e-5)
    norm_sum = jnp.sum(1.0 - jnp.sum(pnu * gnu, axis=1))

    pT = predT_ref[...]
    ab2 = jax.lax.dot_general(p3, pT, (((1,), (0,)), ((), ())),
                              preferred_element_type=jnp.float32)
    b2p = jnp.sum(pT * pT, axis=0, keepdims=True)
    sq2 = a2 + b2p - 2.0 * ab2
    colN = jax.lax.broadcasted_iota(jnp.int32, (R, N), 1)
    rowg = i * R + jax.lax.broadcasted_iota(jnp.int32, (R, N), 0)
    sq2 = jnp.where(colN == rowg, jnp.inf, sq2)
    md = jnp.sqrt(jnp.maximum(jnp.min(sq2, axis=1), 1e-12))
    pen = _softplus(100.0 * (0.3 - md))
    rep = jnp.sum(pen * pen)

    partial = attr / (N * 3.0) + rep / N + 10.0 * norm_sum / N

    @pl.when(i == 0)
    def _first():
        out_ref[0, 0] = partial

    @pl.when(i != 0)
    def _rest():
        out_ref[0, 0] = out_ref[0, 0] + partial


def kernel(pred_feat, pred_decoder, input_data, gt_data):
    del pred_decoder, input_data
    pp = pred_feat[:, :3]
    gtT = gt_data[:, :3].T
    predT = pp.T

    out = pl.pallas_call(
        _loss_kernel,
        grid=(NBLK,),
        in_specs=[
            pl.BlockSpec((R, 3), lambda i: (i, 0)),
            pl.BlockSpec((R, 6), lambda i: (i, 0)),
            pl.BlockSpec((L, 6), lambda i: (0, 0)),
            pl.BlockSpec((3, L), lambda i: (0, 0)),
            pl.BlockSpec((3, N), lambda i: (0, 0)),
        ],
        out_specs=pl.BlockSpec(memory_space=pltpu.SMEM),
        out_shape=jax.ShapeDtypeStruct((1, 1), jnp.float32),
    )(pp, pred_feat, gt_data, gtT, predT)
    return out[0, 0]


# speedup vs baseline: 2.2645x; 2.2645x over previous
import jax
import jax.numpy as jnp
from jax.experimental import pallas as pl
from jax.experimental.pallas import tpu as pltpu

N = 4096
L = 16384
R = 512
C = 2048
NBLK = N // R
CBLK = L // C


def _softplus(x):
    return jnp.maximum(x, 0.0) + jnp.log1p(jnp.exp(-jnp.abs(x)))


def _loss_kernel(p3_ref, pf_ref, gt_ref, gtT_ref, predT_ref, out_ref):
    i = pl.program_id(0)

    p3 = p3_ref[...]
    p3x2 = p3 + p3

    def nn_body(c, carry):
        run_min, run_vals = carry
        g3 = gtT_ref[:, pl.ds(c * C, C)]
        ab2 = jax.lax.dot_general(p3x2, g3, (((1,), (0,)), ((), ())),
                                  preferred_element_type=jnp.float32)
        t = jnp.sum(g3 * g3, axis=0, keepdims=True) - ab2
        bmin = jnp.min(t, axis=1, keepdims=True)
        onehot = (t <= bmin).astype(jnp.float32)
        g6 = gt_ref[pl.ds(c * C, C), :]
        wvals = jax.lax.dot_general(onehot, g6, (((1,), (0,)), ((), ())),
                                    preferred_element_type=jnp.float32)
        better = bmin < run_min
        return (jnp.where(better, bmin, run_min),
                jnp.where(better, wvals, run_vals))

    init = (jnp.full((R, 1), jnp.inf, jnp.float32),
            jnp.zeros((R, 6), jnp.float32))
    _, closest = jax.lax.fori_loop(0, CBLK, nn_body, init)

    diff = p3 - closest[:, 0:3]
    attr = jnp.sum(diff * diff)

    pn = pf_ref[:, 3:6]
    pnu = pn / jnp.maximum(jnp.sqrt(jnp.sum(pn * pn, axis=1, keepdims=True)),
                           1e-5)
    gn = closest[:, 3:6]
    gnu = gn / jnp.maximum(jnp.sqrt(jnp.sum(gn * gn, axis=1, keepdims=True)),
                           1e-5)
    norm_sum = jnp.sum(1.0 - jnp.sum(pnu * gnu, axis=1))

    pT = predT_ref[...]
    ab2r = jax.lax.dot_general(p3x2, pT, (((1,), (0,)), ((), ())),
                               preferred_element_type=jnp.float32)
    b2p = jnp.sum(pT * pT, axis=0, keepdims=True)
    t2 = b2p - ab2r
    colN = jax.lax.broadcasted_iota(jnp.int32, (R, N), 1)
    rowg = i * R + jax.lax.broadcasted_iota(jnp.int32, (R, N), 0)
    t2 = jnp.where(colN == rowg, jnp.inf, t2)
    a2 = jnp.sum(p3 * p3, axis=1)
    md = jnp.sqrt(jnp.maximum(a2 + jnp.min(t2, axis=1), 1e-12))
    pen = _softplus(100.0 * (0.3 - md))
    rep = jnp.sum(pen * pen)

    partial = attr / (N * 3.0) + rep / N + 10.0 * norm_sum / N

    @pl.when(i == 0)
    def _first():
        out_ref[0, 0] = partial

    @pl.when(i != 0)
    def _rest():
        out_ref[0, 0] = out_ref[0, 0] + partial


def kernel(pred_feat, pred_decoder, input_data, gt_data):
    del pred_decoder, input_data
    pp = pred_feat[:, :3]
    gtT = gt_data[:, :3].T
    predT = pp.T

    out = pl.pallas_call(
        _loss_kernel,
        grid=(NBLK,),
        in_specs=[
            pl.BlockSpec((R, 3), lambda i: (i, 0)),
            pl.BlockSpec((R, 6), lambda i: (i, 0)),
            pl.BlockSpec((L, 6), lambda i: (0, 0)),
            pl.BlockSpec((3, L), lambda i: (0, 0)),
            pl.BlockSpec((3, N), lambda i: (0, 0)),
        ],
        out_specs=pl.BlockSpec(memory_space=pltpu.SMEM),
        out_shape=jax.ShapeDtypeStruct((1, 1), jnp.float32),
    )(pp, pred_feat, gt_data, gtT, predT)
    return out[0, 0]


# speedup vs baseline: 2.2795x; 1.0066x over previous
import jax
import jax.numpy as jnp
from jax.experimental import pallas as pl
from jax.experimental.pallas import tpu as pltpu

N = 4096
L = 16384
R = 512
C = 4096
NBLK = N // R
CBLK = L // C


def _softplus(x):
    return jnp.maximum(x, 0.0) + jnp.log1p(jnp.exp(-jnp.abs(x)))


def _loss_kernel(p3_ref, pf_ref, gt_ref, gtT_ref, predT_ref, out_ref):
    i = pl.program_id(0)

    p3 = p3_ref[...]
    p3x2 = p3 + p3

    def nn_body(c, carry):
        run_min, run_vals = carry
        g3 = gtT_ref[:, pl.ds(c * C, C)]
        ab2 = jax.lax.dot_general(p3x2, g3, (((1,), (0,)), ((), ())),
                                  preferred_element_type=jnp.float32)
        t = jnp.sum(g3 * g3, axis=0, keepdims=True) - ab2
        bmin = jnp.min(t, axis=1, keepdims=True)
        onehot = (t <= bmin).astype(jnp.bfloat16)
        g6 = gt_ref[pl.ds(c * C, C), :]
        wvals = jax.lax.dot_general(onehot, g6, (((1,), (0,)), ((), ())),
                                    preferred_element_type=jnp.float32)
        better = bmin < run_min
        return (jnp.where(better, bmin, run_min),
                jnp.where(better, wvals, run_vals))

    init = (jnp.full((R, 1), jnp.inf, jnp.float32),
            jnp.zeros((R, 6), jnp.float32))
    _, closest = jax.lax.fori_loop(0, CBLK, nn_body, init)

    diff = p3 - closest[:, 0:3]
    attr = jnp.sum(diff * diff)

    pn = pf_ref[:, 3:6]
    pnu = pn / jnp.maximum(jnp.sqrt(jnp.sum(pn * pn, axis=1, keepdims=True)),
                           1e-5)
    gn = closest[:, 3:6]
    gnu = gn / jnp.maximum(jnp.sqrt(jnp.sum(gn * gn, axis=1, keepdims=True)),
                           1e-5)
    norm_sum = jnp.sum(1.0 - jnp.sum(pnu * gnu, axis=1))

    pT = predT_ref[...]
    ab2r = jax.lax.dot_general(p3x2, pT, (((1,), (0,)), ((), ())),
                               preferred_element_type=jnp.float32)
    b2p = jnp.sum(pT * pT, axis=0, keepdims=True)
    t2 = b2p - ab2r
    colN = jax.lax.broadcasted_iota(jnp.int32, (R, N), 1)
    rowg = i * R + jax.lax.broadcasted_iota(jnp.int32, (R, 1), 0)
    t2 = jnp.where(colN == rowg, jnp.inf, t2)
    a2 = jnp.sum(p3 * p3, axis=1)
    md = jnp.sqrt(jnp.maximum(a2 + jnp.min(t2, axis=1), 1e-12))
    pen = _softplus(100.0 * (0.3 - md))
    rep = jnp.sum(pen * pen)

    partial = attr / (N * 3.0) + rep / N + 10.0 * norm_sum / N

    @pl.when(i == 0)
    def _first():
        out_ref[0, 0] = partial

    @pl.when(i != 0)
    def _rest():
        out_ref[0, 0] = out_ref[0, 0] + partial


def kernel(pred_feat, pred_decoder, input_data, gt_data):
    del pred_decoder, input_data
    pp = pred_feat[:, :3]
    gtT = gt_data[:, :3].T
    predT = pp.T

    out = pl.pallas_call(
        _loss_kernel,
        grid=(NBLK,),
        in_specs=[
            pl.BlockSpec((R, 3), lambda i: (i, 0)),
            pl.BlockSpec((R, 6), lambda i: (i, 0)),
            pl.BlockSpec((L, 6), lambda i: (0, 0)),
            pl.BlockSpec((3, L), lambda i: (0, 0)),
            pl.BlockSpec((3, N), lambda i: (0, 0)),
        ],
        out_specs=pl.BlockSpec(memory_space=pltpu.SMEM),
        out_shape=jax.ShapeDtypeStruct((1, 1), jnp.float32),
    )(pp, pred_feat, gt_data.astype(jnp.bfloat16), gtT, predT)
    return out[0, 0]


# speedup vs baseline: 2.4108x; 1.0576x over previous
import jax
import jax.numpy as jnp
from jax.experimental import pallas as pl
from jax.experimental.pallas import tpu as pltpu

N = 4096
L = 16384
R = 512
C = 4096
NBLK = N // R
CBLK = L // C


def _softplus(x):
    return jnp.maximum(x, 0.0) + jnp.log1p(jnp.exp(-jnp.abs(x)))


def _split3(x):
    hi = x.astype(jnp.bfloat16).astype(jnp.float32)
    r1 = x - hi
    mid = r1.astype(jnp.bfloat16).astype(jnp.float32)
    lo = (r1 - mid).astype(jnp.bfloat16).astype(jnp.float32)
    return hi, mid, lo


def _loss_kernel(p3_ref, pf_ref, gt_ref, gtT_ref, predT_ref, out_ref,
                 gta_ref, pra_ref):
    i = pl.program_id(0)

    @pl.when(i == 0)
    def _init():
        g = gtT_ref[...]
        gta_ref[0:3, :] = g
        hi, mid, lo = _split3(jnp.sum(g * g, axis=0, keepdims=True))
        gta_ref[3:4, :] = hi
        gta_ref[4:5, :] = mid
        gta_ref[5:6, :] = lo
        p = predT_ref[...]
        pra_ref[0:3, :] = p
        hi2, mid2, lo2 = _split3(jnp.sum(p * p, axis=0, keepdims=True))
        pra_ref[3:4, :] = hi2
        pra_ref[4:5, :] = mid2
        pra_ref[5:6, :] = lo2

    p3 = p3_ref[...]
    lhs = jnp.concatenate([-(p3 + p3), jnp.ones((R, 3), jnp.float32)],
                          axis=1)

    def nn_body(c, carry):
        run_min, run_vals = carry
        ga = gta_ref[:, pl.ds(c * C, C)]
        t = jax.lax.dot_general(lhs, ga, (((1,), (0,)), ((), ())),
                                preferred_element_type=jnp.float32)
        bmin = jnp.min(t, axis=1, keepdims=True)
        onehot = (t <= bmin).astype(jnp.bfloat16)
        g6 = gt_ref[pl.ds(c * C, C), :]
        wvals = jax.lax.dot_general(onehot, g6, (((1,), (0,)), ((), ())),
                                    preferred_element_type=jnp.float32)
        better = bmin < run_min
        return (jnp.where(better, bmin, run_min),
                jnp.where(better, wvals, run_vals))

    init = (jnp.full((R, 1), jnp.inf, jnp.float32),
            jnp.zeros((R, 6), jnp.float32))
    _, closest = jax.lax.fori_loop(0, CBLK, nn_body, init)

    diff = p3 - closest[:, 0:3]
    attr = jnp.sum(diff * diff)

    pn = pf_ref[:, 3:6]
    pnu = pn / jnp.maximum(jnp.sqrt(jnp.sum(pn * pn, axis=1, keepdims=True)),
                           1e-5)
    gn = closest[:, 3:6]
    gnu = gn / jnp.maximum(jnp.sqrt(jnp.sum(gn * gn, axis=1, keepdims=True)),
                           1e-5)
    norm_sum = jnp.sum(1.0 - jnp.sum(pnu * gnu, axis=1))

    t2 = jax.lax.dot_general(lhs, pra_ref[...], (((1,), (0,)), ((), ())),
                             preferred_element_type=jnp.float32)
    colN = jax.lax.broadcasted_iota(jnp.int32, (R, N), 1)
    rowg = i * R + jax.lax.broadcasted_iota(jnp.int32, (R, 1), 0)
    t2 = jnp.where(colN == rowg, jnp.inf, t2)
    a2 = jnp.sum(p3 * p3, axis=1)
    md = jnp.sqrt(jnp.maximum(a2 + jnp.min(t2, axis=1), 1e-12))
    pen = _softplus(100.0 * (0.3 - md))
    rep = jnp.sum(pen * pen)

    partial = attr / (N * 3.0) + rep / N + 10.0 * norm_sum / N

    @pl.when(i == 0)
    def _first():
        out_ref[0, 0] = partial

    @pl.when(i != 0)
    def _rest():
        out_ref[0, 0] = out_ref[0, 0] + partial


def kernel(pred_feat, pred_decoder, input_data, gt_data):
    del pred_decoder, input_data
    pp = pred_feat[:, :3]
    gtT = gt_data[:, :3].T
    predT = pp.T

    out = pl.pallas_call(
        _loss_kernel,
        grid=(NBLK,),
        in_specs=[
            pl.BlockSpec((R, 3), lambda i: (i, 0)),
            pl.BlockSpec((R, 6), lambda i: (i, 0)),
            pl.BlockSpec((L, 6), lambda i: (0, 0)),
            pl.BlockSpec((3, L), lambda i: (0, 0)),
            pl.BlockSpec((3, N), lambda i: (0, 0)),
        ],
        out_specs=pl.BlockSpec(memory_space=pltpu.SMEM),
        out_shape=jax.ShapeDtypeStruct((1, 1), jnp.float32),
        scratch_shapes=[
            pltpu.VMEM((6, L), jnp.float32),
            pltpu.VMEM((6, N), jnp.float32),
        ],
    )(pp, pred_feat, gt_data.astype(jnp.bfloat16), gtT, predT)
    return out[0, 0]


# speedup vs baseline: 3.5475x; 1.4715x over previous
import jax
import jax.numpy as jnp
from jax.experimental import pallas as pl
from jax.experimental.pallas import tpu as pltpu

N = 4096
L = 16384
R = 512
C = 4096
NBLK = N // R
CBLK = L // C


def _softplus(x):
    return jnp.maximum(x, 0.0) + jnp.log1p(jnp.exp(-jnp.abs(x)))


def _split3(x):
    hi = x.astype(jnp.bfloat16).astype(jnp.float32)
    r1 = x - hi
    mid = r1.astype(jnp.bfloat16).astype(jnp.float32)
    lo = (r1 - mid).astype(jnp.bfloat16).astype(jnp.float32)
    return hi, mid, lo


def _loss_kernel(p3_ref, pf_ref, gt_ref, gtT_ref, predT_ref, out_ref,
                 gta_ref, pra_ref):
    i = pl.program_id(0)

    @pl.when(i == 0)
    def _init():
        g = gtT_ref[...]
        gta_ref[0:3, :] = g
        hi, mid, lo = _split3(jnp.sum(g * g, axis=0, keepdims=True))
        gta_ref[3:4, :] = hi
        gta_ref[4:5, :] = mid
        gta_ref[5:6, :] = lo
        p = predT_ref[...]
        pra_ref[0:3, :] = p
        hi2, mid2, lo2 = _split3(jnp.sum(p * p, axis=0, keepdims=True))
        pra_ref[3:4, :] = hi2
        pra_ref[4:5, :] = mid2
        pra_ref[5:6, :] = lo2

    p3 = p3_ref[...]
    lhs = jnp.concatenate([-(p3 + p3), jnp.ones((R, 3), jnp.float32)],
                          axis=1)

    run_min = None
    run_vals = None
    for c in range(CBLK):
        ga = gta_ref[:, c * C:(c + 1) * C]
        t = jax.lax.dot_general(lhs, ga, (((1,), (0,)), ((), ())),
                                preferred_element_type=jnp.float32)
        bmin = jnp.min(t, axis=1, keepdims=True)
        onehot = (t <= bmin).astype(jnp.bfloat16)
        g6 = gt_ref[c * C:(c + 1) * C, :]
        wvals = jax.lax.dot_general(onehot, g6, (((1,), (0,)), ((), ())),
                                    preferred_element_type=jnp.float32)
        if run_min is None:
            run_min, run_vals = bmin, wvals
        else:
            better = bmin < run_min
            run_min = jnp.where(better, bmin, run_min)
            run_vals = jnp.where(better, wvals, run_vals)
    closest = run_vals

    diff = p3 - closest[:, 0:3]
    attr = jnp.sum(diff * diff)

    pn = pf_ref[:, 3:6]
    pnu = pn / jnp.maximum(jnp.sqrt(jnp.sum(pn * pn, axis=1, keepdims=True)),
                           1e-5)
    gn = closest[:, 3:6]
    gnu = gn / jnp.maximum(jnp.sqrt(jnp.sum(gn * gn, axis=1, keepdims=True)),
                           1e-5)
    norm_sum = jnp.sum(1.0 - jnp.sum(pnu * gnu, axis=1))

    t2 = jax.lax.dot_general(lhs, pra_ref[...], (((1,), (0,)), ((), ())),
                             preferred_element_type=jnp.float32)
    colN = jax.lax.broadcasted_iota(jnp.int32, (R, N), 1)
    rowg = i * R + jax.lax.broadcasted_iota(jnp.int32, (R, 1), 0)
    t2 = jnp.where(colN == rowg, jnp.inf, t2)
    a2 = jnp.sum(p3 * p3, axis=1)
    md = jnp.sqrt(jnp.maximum(a2 + jnp.min(t2, axis=1), 1e-12))
    pen = _softplus(100.0 * (0.3 - md))
    rep = jnp.sum(pen * pen)

    partial = attr / (N * 3.0) + rep / N + 10.0 * norm_sum / N

    @pl.when(i == 0)
    def _first():
        out_ref[0, 0] = partial

    @pl.when(i != 0)
    def _rest():
        out_ref[0, 0] = out_ref[0, 0] + partial


def kernel(pred_feat, pred_decoder, input_data, gt_data):
    del pred_decoder, input_data
    pp = pred_feat[:, :3]
    gtT = gt_data[:, :3].T
    predT = pp.T

    out = pl.pallas_call(
        _loss_kernel,
        grid=(NBLK,),
        in_specs=[
            pl.BlockSpec((R, 3), lambda i: (i, 0)),
            pl.BlockSpec((R, 6), lambda i: (i, 0)),
            pl.BlockSpec((L, 6), lambda i: (0, 0)),
            pl.BlockSpec((3, L), lambda i: (0, 0)),
            pl.BlockSpec((3, N), lambda i: (0, 0)),
        ],
        out_specs=pl.BlockSpec(memory_space=pltpu.SMEM),
        out_shape=jax.ShapeDtypeStruct((1, 1), jnp.float32),
        scratch_shapes=[
            pltpu.VMEM((6, L), jnp.float32),
            pltpu.VMEM((6, N), jnp.float32),
        ],
    )(pp, pred_feat, gt_data.astype(jnp.bfloat16), gtT, predT)
    return out[0, 0]
